# Initial kernel scaffold; baseline (speedup 1.0000x reference)
import jax
import jax.numpy as jnp
from jax.experimental import pallas as pl


def kernel(node_features, edge_index, W1, b1, W2, b2, W3, b3):
    raise NotImplementedError("write your pallas kernel here")


# speedup vs baseline: 15.2394x; 15.2394x over previous
import jax
import jax.numpy as jnp
from jax import lax
from jax.experimental import pallas as pl
from jax.experimental.pallas import tpu as pltpu
from jax.experimental.pallas import tpu_sc as plsc

_NC = 2
_NS = 16
_NW = _NC * _NS

_HALF = 5120
_FULL = 2 * _HALF

_RB = 1000


def _sc_mesh():
    return plsc.VectorSubcoreMesh(core_axis_name="c", subcore_axis_name="s",
                                  num_cores=_NC, num_subcores=_NS)


def _deg_body(dst_hbm, zeros_hbm, out_hbm, dst_v, hist_v):
    cid = lax.axis_index("c")
    sid = lax.axis_index("s")
    wid = sid * _NC + cid
    epw = dst_v.shape[0]
    hwords = _HALF * 16

    pltpu.sync_copy(dst_hbm.at[wid], dst_v)

    lane = lax.iota(jnp.int32, 16)
    ones = jnp.ones((16,), jnp.float32)

    for p in range(2):
        lo = p * _HALF
        pltpu.sync_copy(zeros_hbm, hist_v)

        def body(i, carry, lo=lo):
            dvec = dst_v[pl.ds(i * 16, 16)]
            m = (dvec >= lo) & (dvec < lo + _HALF)
            idx = jnp.where(m, dvec - lo, 0) * 16 + lane
            plsc.addupdate_scatter(hist_v, [idx], ones, mask=m)
            return carry

        lax.fori_loop(0, epw // 16, body, 0)
        pltpu.sync_copy(hist_v, out_hbm.at[wid, pl.ds(p * hwords, hwords)])


def _acc_body(gl_hbm, gr_hbm, src_hbm, dst_hbm, zeros_hbm, out_hbm,
              src_v, dst_v, buf0, buf1, sem0, sem1, shared_acc):
    npad = shared_acc.shape[0]
    nchunk = src_v.shape[0]
    cid = lax.axis_index("c")
    sid = lax.axis_index("s")
    rpt = npad // _NS
    r0 = pl.multiple_of(sid * rpt, 8)

    pltpu.sync_copy(src_hbm.at[sid], src_v)
    pltpu.sync_copy(dst_hbm.at[sid], dst_v)
    pltpu.sync_copy(zeros_hbm.at[pl.ds(r0, rpt)], shared_acc.at[pl.ds(r0, rpt)])
    plsc.subcore_barrier()

    bufs = (buf0, buf1)
    sems = (sem0, sem1)

    def run(table):
        pltpu.async_copy(table.at[src_v.at[0]], buf0, sem0)
        pltpu.async_copy(table.at[src_v.at[1]], buf1, sem1)

        def outer(gidx, carry):
            for b in range(2):
                c = gidx * 2 + b
                buf, sem = bufs[b], sems[b]
                pltpu.make_async_copy(table.at[src_v.at[c]], buf, sem).wait()
                pltpu.sync_copy(buf, shared_acc.at[dst_v.at[c]], add=True)

                @pl.when(c + 2 < nchunk)
                def _(buf=buf, sem=sem, c=c):
                    pltpu.async_copy(table.at[src_v.at[c + 2]], buf, sem)
            return carry

        lax.fori_loop(0, nchunk // 2, outer, 0)

    @pl.when(cid == 0)
    def _():
        run(gl_hbm)

    @pl.when(cid == 1)
    def _():
        run(gr_hbm)

    plsc.subcore_barrier()
    pltpu.sync_copy(shared_acc.at[pl.ds(r0, rpt)],
                    out_hbm.at[cid, pl.ds(r0, rpt)])


def _make_acc_call(dc, nchunk, ch):
    return pl.kernel(
        _acc_body,
        out_type=jax.ShapeDtypeStruct((_NC, _FULL, dc), jnp.float32),
        mesh=_sc_mesh(),
        scratch_types=[
            pltpu.VMEM((nchunk, ch), jnp.int32),
            pltpu.VMEM((nchunk, ch), jnp.int32),
            pltpu.VMEM((ch, dc), jnp.float32),
            pltpu.VMEM((ch, dc), jnp.float32),
            pltpu.SemaphoreType.DMA,
            pltpu.SemaphoreType.DMA,
            pltpu.VMEM_SHARED((_FULL, dc), jnp.float32),
        ],
        compiler_params=pltpu.CompilerParams(needs_layout_passes=False,
                                             use_tc_tiling_on_sc=False),
    )


def _tc_first_body(deg_ref, x_ref, w_ref, g_ref, inv_ref):
    deg = jnp.sum(deg_ref[...], axis=(0, 2)) + 1.0
    inv = lax.rsqrt(deg)[:, None]
    inv_ref[...] = inv
    g_ref[...] = jnp.dot(x_ref[...] * inv, w_ref[...],
                         preferred_element_type=jnp.float32)


def _tc_mid_body(acc_ref, g_ref, inv_ref, b_ref, w_ref, o_ref):
    inv = inv_ref[...]
    acc = jnp.concatenate((acc_ref[0], acc_ref[1]), axis=-1)
    x = inv * (acc + g_ref[...]) + b_ref[...]
    x = jnp.maximum(x, 0.0)
    o_ref[...] = jnp.dot(x * inv, w_ref[...],
                         preferred_element_type=jnp.float32)


def _tc_final_body(acc_ref, g_ref, inv_ref, b_ref, o_ref):
    acc = jnp.concatenate((acc_ref[0], acc_ref[1]), axis=-1)
    o_ref[...] = inv_ref[...] * (acc + g_ref[...]) + b_ref[...]


def _tc_first(deg_p, x, W):
    n, d_in = x.shape
    d_out = W.shape[1]
    return pl.pallas_call(
        _tc_first_body,
        grid=(n // _RB,),
        in_specs=[
            pl.BlockSpec((_NW, _RB, 16), lambda i: (0, i, 0)),
            pl.BlockSpec((_RB, d_in), lambda i: (i, 0)),
            pl.BlockSpec((d_in, d_out), lambda i: (0, 0)),
        ],
        out_specs=[
            pl.BlockSpec((_RB, d_out), lambda i: (i, 0)),
            pl.BlockSpec((_RB, 1), lambda i: (i, 0)),
        ],
        out_shape=[
            jax.ShapeDtypeStruct((n, d_out), jnp.float32),
            jax.ShapeDtypeStruct((n, 1), jnp.float32),
        ],
    )(deg_p, x, W)


def _tc_mid(acc_p, g, inv, b, W):
    n, d = g.shape
    d_out = W.shape[1]
    return pl.pallas_call(
        _tc_mid_body,
        grid=(n // _RB,),
        in_specs=[
            pl.BlockSpec((_NC, _RB, d // 2), lambda i: (0, i, 0)),
            pl.BlockSpec((_RB, d), lambda i: (i, 0)),
            pl.BlockSpec((_RB, 1), lambda i: (i, 0)),
            pl.BlockSpec((1, d), lambda i: (0, 0)),
            pl.BlockSpec((d, d_out), lambda i: (0, 0)),
        ],
        out_specs=pl.BlockSpec((_RB, d_out), lambda i: (i, 0)),
        out_shape=jax.ShapeDtypeStruct((n, d_out), jnp.float32),
    )(acc_p, g, inv, b, W)


def _tc_final(acc_p, g, inv, b):
    n, d = g.shape
    return pl.pallas_call(
        _tc_final_body,
        grid=(n // _RB,),
        in_specs=[
            pl.BlockSpec((_NC, _RB, d // 2), lambda i: (0, i, 0)),
            pl.BlockSpec((_RB, d), lambda i: (i, 0)),
            pl.BlockSpec((_RB, 1), lambda i: (i, 0)),
            pl.BlockSpec((1, d), lambda i: (0, 0)),
        ],
        out_specs=pl.BlockSpec((_RB, d), lambda i: (i, 0)),
        out_shape=jax.ShapeDtypeStruct((n, d), jnp.float32),
    )(acc_p, g, inv, b)


def kernel(node_features, edge_index, W1, b1, W2, b2, W3, b3):
    n, d_in = node_features.shape
    e = edge_index.shape[1]
    epw = e // _NW
    ch = 100
    nchunk = (e // _NS) // ch

    src = edge_index[0].reshape(_NS, nchunk, ch)
    dst = edge_index[1].reshape(_NS, nchunk, ch)
    dst_flat = edge_index[1].reshape(_NW, epw)
    zeros_deg = jnp.zeros((_HALF * 16,), jnp.float32)
    zeros_h = jnp.zeros((_FULL, W1.shape[1] // 2), jnp.float32)
    zeros_o = jnp.zeros((_FULL, W3.shape[1] // 2), jnp.float32)

    deg_call = pl.kernel(
        _deg_body,
        out_type=jax.ShapeDtypeStruct((_NW, _FULL * 16), jnp.float32),
        mesh=_sc_mesh(),
        scratch_types=[
            pltpu.VMEM((epw,), jnp.int32),
            pltpu.VMEM((_HALF * 16,), jnp.float32),
        ],
        compiler_params=pltpu.CompilerParams(needs_layout_passes=False),
    )
    deg_p = deg_call(dst_flat, zeros_deg).reshape(_NW, _FULL, 16)

    dh = W1.shape[1] // 2
    do = W3.shape[1] // 2
    acc_h = _make_acc_call(dh, nchunk, ch)
    acc_o = _make_acc_call(do, nchunk, ch)

    g1, inv = _tc_first(deg_p, node_features, W1)
    a1 = acc_h(g1[:, :dh], g1[:, dh:], src, dst, zeros_h)
    g2 = _tc_mid(a1, g1, inv, b1.reshape(1, -1), W2)
    a2 = acc_h(g2[:, :dh], g2[:, dh:], src, dst, zeros_h)
    g3 = _tc_mid(a2, g2, inv, b2.reshape(1, -1), W3)
    a3 = acc_o(g3[:, :do], g3[:, do:], src, dst, zeros_o)
    return _tc_final(a3, g3, inv, b3.reshape(1, -1))


# speedup vs baseline: 21.0416x; 1.3807x over previous
import jax
import jax.numpy as jnp
from jax import lax
from jax.experimental import pallas as pl
from jax.experimental.pallas import tpu as pltpu
from jax.experimental.pallas import tpu_sc as plsc

_NC = 2
_NS = 16
_NW = _NC * _NS

_HALF = 5120
_FULL = 2 * _HALF

_RB = 1000


def _sc_mesh():
    return plsc.VectorSubcoreMesh(core_axis_name="c", subcore_axis_name="s",
                                  num_cores=_NC, num_subcores=_NS)


def _deg_body(dst_hbm, zeros_hbm, out_hbm, dst_v, hist_v, deg_v):
    cid = lax.axis_index("c")
    sid = lax.axis_index("s")
    wid = sid * _NC + cid
    epw = dst_v.shape[0]

    pltpu.sync_copy(dst_hbm.at[wid], dst_v)

    lane_base = lax.iota(jnp.int32, 16) * _HALF
    ones = jnp.ones((16,), jnp.float32)

    for p in range(2):
        lo = p * _HALF
        pltpu.sync_copy(zeros_hbm, hist_v)

        def body(i, carry, lo=lo):
            dvec = dst_v[pl.ds(i * 16, 16)]
            m = (dvec >= lo) & (dvec < lo + _HALF)
            idx = jnp.where(m, dvec - lo, 0) + lane_base
            plsc.addupdate_scatter(hist_v, [idx], ones, mask=m)
            return carry

        lax.fori_loop(0, epw // 16, body, 0)

        def red(q, carry):
            s = hist_v[pl.ds(q * 16, 16)]
            for l in range(1, 16):
                s = s + hist_v[pl.ds(l * _HALF + q * 16, 16)]
            deg_v[pl.ds(q * 16, 16)] = s
            return carry

        lax.fori_loop(0, _HALF // 16, red, 0)
        pltpu.sync_copy(deg_v, out_hbm.at[wid, pl.ds(p * _HALF, _HALF)])


def _acc_body(gl_hbm, gr_hbm, src_hbm, dst_hbm, zeros_hbm, out_hbm,
              src_v, dst_v, buf0, buf1, sem0, sem1, shared_acc):
    npad = shared_acc.shape[0]
    nchunk = src_v.shape[0]
    cid = lax.axis_index("c")
    sid = lax.axis_index("s")
    rpt = npad // _NS
    r0 = pl.multiple_of(sid * rpt, 8)

    pltpu.sync_copy(src_hbm.at[sid], src_v)
    pltpu.sync_copy(dst_hbm.at[sid], dst_v)
    pltpu.sync_copy(zeros_hbm.at[pl.ds(r0, rpt)], shared_acc.at[pl.ds(r0, rpt)])
    plsc.subcore_barrier()

    bufs = (buf0, buf1)
    sems = (sem0, sem1)

    def run(table):
        pltpu.async_copy(table.at[src_v.at[0]], buf0, sem0)
        pltpu.async_copy(table.at[src_v.at[1]], buf1, sem1)

        def outer(gidx, carry):
            for b in range(2):
                c = gidx * 2 + b
                buf, sem = bufs[b], sems[b]
                pltpu.make_async_copy(table.at[src_v.at[c]], buf, sem).wait()
                pltpu.sync_copy(buf, shared_acc.at[dst_v.at[c]], add=True)

                @pl.when(c + 2 < nchunk)
                def _(buf=buf, sem=sem, c=c):
                    pltpu.async_copy(table.at[src_v.at[c + 2]], buf, sem)
            return carry

        lax.fori_loop(0, nchunk // 2, outer, 0)

    @pl.when(cid == 0)
    def _():
        run(gl_hbm)

    @pl.when(cid == 1)
    def _():
        run(gr_hbm)

    plsc.subcore_barrier()
    pltpu.sync_copy(shared_acc.at[pl.ds(r0, rpt)],
                    out_hbm.at[cid, pl.ds(r0, rpt)])


def _make_acc_call(dc, nchunk, ch):
    return pl.kernel(
        _acc_body,
        out_type=jax.ShapeDtypeStruct((_NC, _FULL, dc), jnp.float32),
        mesh=_sc_mesh(),
        scratch_types=[
            pltpu.VMEM((nchunk, ch), jnp.int32),
            pltpu.VMEM((nchunk, ch), jnp.int32),
            pltpu.VMEM((ch, dc), jnp.float32),
            pltpu.VMEM((ch, dc), jnp.float32),
            pltpu.SemaphoreType.DMA,
            pltpu.SemaphoreType.DMA,
            pltpu.VMEM_SHARED((_FULL, dc), jnp.float32),
        ],
        compiler_params=pltpu.CompilerParams(needs_layout_passes=False,
                                             use_tc_tiling_on_sc=False),
    )


def _tc_first_body(deg_ref, x_ref, w_ref, g_ref, inv_ref):
    deg = jnp.sum(deg_ref[...], axis=1) + 1.0
    inv = lax.rsqrt(deg)[:, None]
    inv_ref[...] = inv
    g_ref[...] = jnp.dot(x_ref[...] * inv, w_ref[...],
                         preferred_element_type=jnp.float32)


def _tc_mid_body(acc_ref, g_ref, inv_ref, b_ref, w_ref, o_ref):
    inv = inv_ref[...]
    acc = jnp.concatenate((acc_ref[0], acc_ref[1]), axis=-1)
    x = inv * (acc + g_ref[...]) + b_ref[...]
    x = jnp.maximum(x, 0.0)
    o_ref[...] = jnp.dot(x * inv, w_ref[...],
                         preferred_element_type=jnp.float32)


def _tc_final_body(acc_ref, g_ref, inv_ref, b_ref, o_ref):
    acc = jnp.concatenate((acc_ref[0], acc_ref[1]), axis=-1)
    o_ref[...] = inv_ref[...] * (acc + g_ref[...]) + b_ref[...]


def _tc_first(deg_p, x, W):
    n, d_in = x.shape
    d_out = W.shape[1]
    return pl.pallas_call(
        _tc_first_body,
        grid=(n // _RB,),
        in_specs=[
            pl.BlockSpec((_RB, _NW), lambda i: (i, 0)),
            pl.BlockSpec((_RB, d_in), lambda i: (i, 0)),
            pl.BlockSpec((d_in, d_out), lambda i: (0, 0)),
        ],
        out_specs=[
            pl.BlockSpec((_RB, d_out), lambda i: (i, 0)),
            pl.BlockSpec((_RB, 1), lambda i: (i, 0)),
        ],
        out_shape=[
            jax.ShapeDtypeStruct((n, d_out), jnp.float32),
            jax.ShapeDtypeStruct((n, 1), jnp.float32),
        ],
    )(deg_p, x, W)


def _tc_mid(acc_p, g, inv, b, W):
    n, d = g.shape
    d_out = W.shape[1]
    return pl.pallas_call(
        _tc_mid_body,
        grid=(n // _RB,),
        in_specs=[
            pl.BlockSpec((_NC, _RB, d // 2), lambda i: (0, i, 0)),
            pl.BlockSpec((_RB, d), lambda i: (i, 0)),
            pl.BlockSpec((_RB, 1), lambda i: (i, 0)),
            pl.BlockSpec((1, d), lambda i: (0, 0)),
            pl.BlockSpec((d, d_out), lambda i: (0, 0)),
        ],
        out_specs=pl.BlockSpec((_RB, d_out), lambda i: (i, 0)),
        out_shape=jax.ShapeDtypeStruct((n, d_out), jnp.float32),
    )(acc_p, g, inv, b, W)


def _tc_final(acc_p, g, inv, b):
    n, d = g.shape
    return pl.pallas_call(
        _tc_final_body,
        grid=(n // _RB,),
        in_specs=[
            pl.BlockSpec((_NC, _RB, d // 2), lambda i: (0, i, 0)),
            pl.BlockSpec((_RB, d), lambda i: (i, 0)),
            pl.BlockSpec((_RB, 1), lambda i: (i, 0)),
            pl.BlockSpec((1, d), lambda i: (0, 0)),
        ],
        out_specs=pl.BlockSpec((_RB, d), lambda i: (i, 0)),
        out_shape=jax.ShapeDtypeStruct((n, d), jnp.float32),
    )(acc_p, g, inv, b)


def kernel(node_features, edge_index, W1, b1, W2, b2, W3, b3):
    n, d_in = node_features.shape
    e = edge_index.shape[1]
    epw = e // _NW
    ch = 100
    nchunk = (e // _NS) // ch

    src = edge_index[0].reshape(_NS, nchunk, ch)
    dst = edge_index[1].reshape(_NS, nchunk, ch)
    dst_flat = edge_index[1].reshape(_NW, epw)
    zeros_deg = jnp.zeros((_HALF * 16,), jnp.float32)
    zeros_h = jnp.zeros((_FULL, W1.shape[1] // 2), jnp.float32)
    zeros_o = jnp.zeros((_FULL, W3.shape[1] // 2), jnp.float32)

    deg_call = pl.kernel(
        _deg_body,
        out_type=jax.ShapeDtypeStruct((_NW, _FULL), jnp.float32),
        mesh=_sc_mesh(),
        scratch_types=[
            pltpu.VMEM((epw,), jnp.int32),
            pltpu.VMEM((_HALF * 16,), jnp.float32),
            pltpu.VMEM((_HALF,), jnp.float32),
        ],
        compiler_params=pltpu.CompilerParams(needs_layout_passes=False),
    )
    deg_p = deg_call(dst_flat, zeros_deg).T

    dh = W1.shape[1] // 2
    do = W3.shape[1] // 2
    acc_h = _make_acc_call(dh, nchunk, ch)
    acc_o = _make_acc_call(do, nchunk, ch)

    g1, inv = _tc_first(deg_p, node_features, W1)
    a1 = acc_h(g1[:, :dh], g1[:, dh:], src, dst, zeros_h)
    g2 = _tc_mid(a1, g1, inv, b1.reshape(1, -1), W2)
    a2 = acc_h(g2[:, :dh], g2[:, dh:], src, dst, zeros_h)
    g3 = _tc_mid(a2, g2, inv, b2.reshape(1, -1), W3)
    a3 = acc_o(g3[:, :do], g3[:, do:], src, dst, zeros_o)
    return _tc_final(a3, g3, inv, b3.reshape(1, -1))


# speedup vs baseline: 23.6530x; 1.1241x over previous
import jax
import jax.numpy as jnp
from jax import lax
from jax.experimental import pallas as pl
from jax.experimental.pallas import tpu as pltpu
from jax.experimental.pallas import tpu_sc as plsc

_NC = 2
_NS = 16
_NW = _NC * _NS

_HALF = 5120
_FULL = 2 * _HALF

_RB = 1000


def _sc_mesh():
    return plsc.VectorSubcoreMesh(core_axis_name="c", subcore_axis_name="s",
                                  num_cores=_NC, num_subcores=_NS)


def _deg_body(dst_hbm, zeros_hbm, out_hbm, dst_v, hist_v, deg_v):
    cid = lax.axis_index("c")
    sid = lax.axis_index("s")
    wid = sid * _NC + cid
    epw = dst_v.shape[0]

    pltpu.sync_copy(dst_hbm.at[wid], dst_v)

    lane_base = lax.iota(jnp.int32, 16) * _HALF
    ones = jnp.ones((16,), jnp.float32)

    for p in range(2):
        lo = p * _HALF
        pltpu.sync_copy(zeros_hbm, hist_v)

        def body(i, carry, lo=lo):
            dvec = dst_v[pl.ds(i * 16, 16)]
            m = (dvec >= lo) & (dvec < lo + _HALF)
            idx = jnp.where(m, dvec - lo, 0) + lane_base
            plsc.addupdate_scatter(hist_v, [idx], ones, mask=m)
            return carry

        lax.fori_loop(0, epw // 16, body, 0)

        def red(q, carry):
            s = hist_v[pl.ds(q * 16, 16)]
            for l in range(1, 16):
                s = s + hist_v[pl.ds(l * _HALF + q * 16, 16)]
            deg_v[pl.ds(q * 16, 16)] = s
            return carry

        lax.fori_loop(0, _HALF // 16, red, 0)
        pltpu.sync_copy(deg_v, out_hbm.at[wid, pl.ds(p * _HALF, _HALF)])


_NBUF = 4


def _acc_body(gl_hbm, gr_hbm, src_hbm, dst_hbm, zeros_hbm, out_hbm,
              src_v, dst_v, bufs, gsems, ssems, shared_acc):
    npad = shared_acc.shape[0]
    nchunk = src_v.shape[0]
    cid = lax.axis_index("c")
    sid = lax.axis_index("s")
    rpt = npad // _NS
    r0 = pl.multiple_of(sid * rpt, 8)

    pltpu.sync_copy(src_hbm.at[sid], src_v)
    pltpu.sync_copy(dst_hbm.at[sid], dst_v)
    pltpu.sync_copy(zeros_hbm.at[pl.ds(r0, rpt)], shared_acc.at[pl.ds(r0, rpt)])
    plsc.subcore_barrier()

    def run(table):
        def gather(c, b):
            pltpu.async_copy(table.at[src_v.at[c]], bufs[b], gsems[b])

        def wait_gather(c, b):
            pltpu.make_async_copy(table.at[src_v.at[c]], bufs[b],
                                  gsems[b]).wait()

        def scatter(c, b):
            pltpu.async_copy(bufs[b], shared_acc.at[dst_v.at[c]], ssems[b],
                             add=True)

        def wait_scatter(c, b):
            pltpu.make_async_copy(bufs[b], shared_acc.at[dst_v.at[c]],
                                  ssems[b]).wait()

        gather(0, 0)
        gather(1, 1)

        def outer(gidx, carry):
            for b in range(_NBUF):
                c = gidx * _NBUF + b
                wait_gather(c, b)
                scatter(c, b)

                @pl.when(c >= 2)
                def _(c=c, b=b):
                    wait_scatter(c - 2, (b + 2) % _NBUF)

                @pl.when(c + 2 < nchunk)
                def _(c=c, b=b):
                    gather(c + 2, (b + 2) % _NBUF)
            return carry

        lax.fori_loop(0, nchunk // _NBUF, outer, 0)
        wait_scatter(nchunk - 2, (nchunk - 2) % _NBUF)
        wait_scatter(nchunk - 1, (nchunk - 1) % _NBUF)

    @pl.when(cid == 0)
    def _():
        run(gl_hbm)

    @pl.when(cid == 1)
    def _():
        run(gr_hbm)

    plsc.subcore_barrier()
    pltpu.sync_copy(shared_acc.at[pl.ds(r0, rpt)],
                    out_hbm.at[cid, pl.ds(r0, rpt)])


def _make_acc_call(dc, nchunk, ch):
    return pl.kernel(
        _acc_body,
        out_type=jax.ShapeDtypeStruct((_NC, _FULL, dc), jnp.float32),
        mesh=_sc_mesh(),
        scratch_types=[
            pltpu.VMEM((nchunk, ch), jnp.int32),
            pltpu.VMEM((nchunk, ch), jnp.int32),
            tuple(pltpu.VMEM((ch, dc), jnp.float32) for _ in range(_NBUF)),
            tuple(pltpu.SemaphoreType.DMA for _ in range(_NBUF)),
            tuple(pltpu.SemaphoreType.DMA for _ in range(_NBUF)),
            pltpu.VMEM_SHARED((_FULL, dc), jnp.float32),
        ],
        compiler_params=pltpu.CompilerParams(needs_layout_passes=False,
                                             use_tc_tiling_on_sc=False),
    )


def _tc_first_body(deg_ref, x_ref, w_ref, g_ref, inv_ref):
    deg = jnp.sum(deg_ref[...], axis=1) + 1.0
    inv = lax.rsqrt(deg)[:, None]
    inv_ref[...] = inv
    g_ref[...] = jnp.dot(x_ref[...] * inv, w_ref[...],
                         preferred_element_type=jnp.float32)


def _tc_mid_body(acc_ref, g_ref, inv_ref, b_ref, w_ref, o_ref):
    inv = inv_ref[...]
    acc = jnp.concatenate((acc_ref[0], acc_ref[1]), axis=-1)
    x = inv * (acc + g_ref[...]) + b_ref[...]
    x = jnp.maximum(x, 0.0)
    o_ref[...] = jnp.dot(x * inv, w_ref[...],
                         preferred_element_type=jnp.float32)


def _tc_final_body(acc_ref, g_ref, inv_ref, b_ref, o_ref):
    acc = jnp.concatenate((acc_ref[0], acc_ref[1]), axis=-1)
    o_ref[...] = inv_ref[...] * (acc + g_ref[...]) + b_ref[...]


def _tc_first(deg_p, x, W):
    n, d_in = x.shape
    d_out = W.shape[1]
    return pl.pallas_call(
        _tc_first_body,
        grid=(n // _RB,),
        in_specs=[
            pl.BlockSpec((_RB, _NW), lambda i: (i, 0)),
            pl.BlockSpec((_RB, d_in), lambda i: (i, 0)),
            pl.BlockSpec((d_in, d_out), lambda i: (0, 0)),
        ],
        out_specs=[
            pl.BlockSpec((_RB, d_out), lambda i: (i, 0)),
            pl.BlockSpec((_RB, 1), lambda i: (i, 0)),
        ],
        out_shape=[
            jax.ShapeDtypeStruct((n, d_out), jnp.float32),
            jax.ShapeDtypeStruct((n, 1), jnp.float32),
        ],
    )(deg_p, x, W)


def _tc_mid(acc_p, g, inv, b, W):
    n, d = g.shape
    d_out = W.shape[1]
    return pl.pallas_call(
        _tc_mid_body,
        grid=(n // _RB,),
        in_specs=[
            pl.BlockSpec((_NC, _RB, d // 2), lambda i: (0, i, 0)),
            pl.BlockSpec((_RB, d), lambda i: (i, 0)),
            pl.BlockSpec((_RB, 1), lambda i: (i, 0)),
            pl.BlockSpec((1, d), lambda i: (0, 0)),
            pl.BlockSpec((d, d_out), lambda i: (0, 0)),
        ],
        out_specs=pl.BlockSpec((_RB, d_out), lambda i: (i, 0)),
        out_shape=jax.ShapeDtypeStruct((n, d_out), jnp.float32),
    )(acc_p, g, inv, b, W)


def _tc_final(acc_p, g, inv, b):
    n, d = g.shape
    return pl.pallas_call(
        _tc_final_body,
        grid=(n // _RB,),
        in_specs=[
            pl.BlockSpec((_NC, _RB, d // 2), lambda i: (0, i, 0)),
            pl.BlockSpec((_RB, d), lambda i: (i, 0)),
            pl.BlockSpec((_RB, 1), lambda i: (i, 0)),
            pl.BlockSpec((1, d), lambda i: (0, 0)),
        ],
        out_specs=pl.BlockSpec((_RB, d), lambda i: (i, 0)),
        out_shape=jax.ShapeDtypeStruct((n, d), jnp.float32),
    )(acc_p, g, inv, b)


def kernel(node_features, edge_index, W1, b1, W2, b2, W3, b3):
    n, d_in = node_features.shape
    e = edge_index.shape[1]
    epw = e // _NW
    ch = 125
    nchunk = (e // _NS) // ch

    src = edge_index[0].reshape(_NS, nchunk, ch)
    dst = edge_index[1].reshape(_NS, nchunk, ch)
    dst_flat = edge_index[1].reshape(_NW, epw)
    zeros_deg = jnp.zeros((_HALF * 16,), jnp.float32)
    zeros_h = jnp.zeros((_FULL, W1.shape[1] // 2), jnp.float32)
    zeros_o = jnp.zeros((_FULL, W3.shape[1] // 2), jnp.float32)

    deg_call = pl.kernel(
        _deg_body,
        out_type=jax.ShapeDtypeStruct((_NW, _FULL), jnp.float32),
        mesh=_sc_mesh(),
        scratch_types=[
            pltpu.VMEM((epw,), jnp.int32),
            pltpu.VMEM((_HALF * 16,), jnp.float32),
            pltpu.VMEM((_HALF,), jnp.float32),
        ],
        compiler_params=pltpu.CompilerParams(needs_layout_passes=False),
    )
    deg_p = deg_call(dst_flat, zeros_deg).T

    dh = W1.shape[1] // 2
    do = W3.shape[1] // 2
    acc_h = _make_acc_call(dh, nchunk, ch)
    acc_o = _make_acc_call(do, nchunk, ch)

    g1, inv = _tc_first(deg_p, node_features, W1)
    a1 = acc_h(g1[:, :dh], g1[:, dh:], src, dst, zeros_h)
    g2 = _tc_mid(a1, g1, inv, b1.reshape(1, -1), W2)
    a2 = acc_h(g2[:, :dh], g2[:, dh:], src, dst, zeros_h)
    g3 = _tc_mid(a2, g2, inv, b2.reshape(1, -1), W3)
    a3 = acc_o(g3[:, :do], g3[:, do:], src, dst, zeros_o)
    return _tc_final(a3, g3, inv, b3.reshape(1, -1))


# speedup vs baseline: 28.4924x; 1.2046x over previous
import jax
import jax.numpy as jnp
from jax import lax
from jax.experimental import pallas as pl
from jax.experimental.pallas import tpu as pltpu
from jax.experimental.pallas import tpu_sc as plsc

_NC = 2
_NS = 16
_NW = _NC * _NS

_HALF = 5120
_FULL = 2 * _HALF

_RB = 1000


def _sc_mesh():
    return plsc.VectorSubcoreMesh(core_axis_name="c", subcore_axis_name="s",
                                  num_cores=_NC, num_subcores=_NS)


def _deg_body(dst_hbm, zeros_hbm, out_hbm, dst_v, hist_v, deg_v):
    cid = lax.axis_index("c")
    sid = lax.axis_index("s")
    wid = sid * _NC + cid
    epw = dst_v.shape[0]

    pltpu.sync_copy(dst_hbm.at[wid], dst_v)

    lane_base = lax.iota(jnp.int32, 16) * _HALF
    ones = jnp.ones((16,), jnp.float32)

    for p in range(2):
        lo = p * _HALF
        pltpu.sync_copy(zeros_hbm, hist_v)

        def body(i, carry, lo=lo):
            dvec = dst_v[pl.ds(i * 16, 16)]
            m = (dvec >= lo) & (dvec < lo + _HALF)
            idx = jnp.where(m, dvec - lo, 0) + lane_base
            plsc.addupdate_scatter(hist_v, [idx], ones, mask=m)
            return carry

        lax.fori_loop(0, epw // 16, body, 0)

        def red(q, carry):
            s = hist_v[pl.ds(q * 16, 16)]
            for l in range(1, 16):
                s = s + hist_v[pl.ds(l * _HALF + q * 16, 16)]
            deg_v[pl.ds(q * 16, 16)] = s
            return carry

        lax.fori_loop(0, _HALF // 16, red, 0)
        pltpu.sync_copy(deg_v, out_hbm.at[wid, pl.ds(p * _HALF, _HALF)])


_NBUF = 4


def _acc_body(gl_hbm, gr_hbm, src_hbm, dst_hbm, zeros_hbm, out_hbm,
              src_v, dst_v, bufs, gsems, ssems, shared_acc):
    npad = shared_acc.shape[0]
    nchunk = src_v.shape[0]
    cid = lax.axis_index("c")
    sid = lax.axis_index("s")
    rpt = npad // _NS
    r0 = pl.multiple_of(sid * rpt, 8)

    pltpu.sync_copy(src_hbm.at[sid], src_v)
    pltpu.sync_copy(dst_hbm.at[sid], dst_v)
    pltpu.sync_copy(zeros_hbm.at[pl.ds(r0, rpt)], shared_acc.at[pl.ds(r0, rpt)])
    plsc.subcore_barrier()

    def run(table):
        def gather(c, b):
            pltpu.async_copy(table.at[src_v.at[c]], bufs[b], gsems[b])

        def wait_gather(c, b):
            pltpu.make_async_copy(table.at[src_v.at[c]], bufs[b],
                                  gsems[b]).wait()

        def scatter(c, b):
            pltpu.async_copy(bufs[b], shared_acc.at[dst_v.at[c]], ssems[b],
                             add=True)

        def wait_scatter(c, b):
            pltpu.make_async_copy(bufs[b], shared_acc.at[dst_v.at[c]],
                                  ssems[b]).wait()

        gather(0, 0)
        gather(1, 1)

        def outer(gidx, carry):
            for b in range(_NBUF):
                c = gidx * _NBUF + b
                wait_gather(c, b)
                scatter(c, b)

                @pl.when(c >= 2)
                def _(c=c, b=b):
                    wait_scatter(c - 2, (b + 2) % _NBUF)

                @pl.when(c + 2 < nchunk)
                def _(c=c, b=b):
                    gather(c + 2, (b + 2) % _NBUF)
            return carry

        lax.fori_loop(0, nchunk // _NBUF, outer, 0)
        wait_scatter(nchunk - 2, (nchunk - 2) % _NBUF)
        wait_scatter(nchunk - 1, (nchunk - 1) % _NBUF)

    @pl.when(cid == 0)
    def _():
        run(gl_hbm)

    @pl.when(cid == 1)
    def _():
        run(gr_hbm)

    plsc.subcore_barrier()
    pltpu.sync_copy(shared_acc.at[pl.ds(r0, rpt)],
                    out_hbm.at[cid, pl.ds(r0, rpt)])


def _make_acc_call(dc, nchunk, ch):
    return pl.kernel(
        _acc_body,
        out_type=jax.ShapeDtypeStruct((_NC, _FULL, dc), jnp.bfloat16),
        mesh=_sc_mesh(),
        scratch_types=[
            pltpu.VMEM((nchunk, ch), jnp.int32),
            pltpu.VMEM((nchunk, ch), jnp.int32),
            tuple(pltpu.VMEM((ch, dc), jnp.bfloat16) for _ in range(_NBUF)),
            tuple(pltpu.SemaphoreType.DMA for _ in range(_NBUF)),
            tuple(pltpu.SemaphoreType.DMA for _ in range(_NBUF)),
            pltpu.VMEM_SHARED((_FULL, dc), jnp.bfloat16),
        ],
        compiler_params=pltpu.CompilerParams(needs_layout_passes=False,
                                             use_tc_tiling_on_sc=False),
    )


def _tc_first_body(deg_ref, x_ref, w_ref, gl_ref, gr_ref, inv_ref):
    dc = gl_ref.shape[1]
    deg = jnp.sum(deg_ref[...], axis=1) + 1.0
    inv = lax.rsqrt(deg)[:, None]
    inv_ref[...] = inv
    h = jnp.dot(x_ref[...] * inv, w_ref[...],
                preferred_element_type=jnp.float32)
    gl_ref[...] = h[:, :dc].astype(jnp.bfloat16)
    gr_ref[...] = h[:, dc:].astype(jnp.bfloat16)


def _tc_mid_body(acc_ref, gl_ref, gr_ref, inv_ref, b_ref, w_ref,
                 ol_ref, or_ref):
    dc = ol_ref.shape[1]
    inv = inv_ref[...]
    left = (acc_ref[0] + gl_ref[...]).astype(jnp.float32)
    right = (acc_ref[1] + gr_ref[...]).astype(jnp.float32)
    x = inv * jnp.concatenate((left, right), axis=-1) + b_ref[...]
    x = jnp.maximum(x, 0.0)
    h = jnp.dot(x * inv, w_ref[...], preferred_element_type=jnp.float32)
    ol_ref[...] = h[:, :dc].astype(jnp.bfloat16)
    or_ref[...] = h[:, dc:].astype(jnp.bfloat16)


def _tc_final_body(acc_ref, gl_ref, gr_ref, inv_ref, b_ref, o_ref):
    left = (acc_ref[0] + gl_ref[...]).astype(jnp.float32)
    right = (acc_ref[1] + gr_ref[...]).astype(jnp.float32)
    o_ref[...] = (inv_ref[...] * jnp.concatenate((left, right), axis=-1)
                  + b_ref[...])


def _tc_first(deg_p, x, W):
    n, d_in = x.shape
    d_out = W.shape[1]
    dc = d_out // 2
    return pl.pallas_call(
        _tc_first_body,
        grid=(n // _RB,),
        in_specs=[
            pl.BlockSpec((_RB, _NW), lambda i: (i, 0)),
            pl.BlockSpec((_RB, d_in), lambda i: (i, 0)),
            pl.BlockSpec((d_in, d_out), lambda i: (0, 0)),
        ],
        out_specs=[
            pl.BlockSpec((_RB, dc), lambda i: (i, 0)),
            pl.BlockSpec((_RB, dc), lambda i: (i, 0)),
            pl.BlockSpec((_RB, 1), lambda i: (i, 0)),
        ],
        out_shape=[
            jax.ShapeDtypeStruct((n, dc), jnp.bfloat16),
            jax.ShapeDtypeStruct((n, dc), jnp.bfloat16),
            jax.ShapeDtypeStruct((n, 1), jnp.float32),
        ],
    )(deg_p, x, W)


def _tc_mid(acc_p, gl, gr, inv, b, W):
    n, dc = gl.shape
    d = 2 * dc
    d_out = W.shape[1]
    dco = d_out // 2
    return pl.pallas_call(
        _tc_mid_body,
        grid=(n // _RB,),
        in_specs=[
            pl.BlockSpec((_NC, _RB, dc), lambda i: (0, i, 0)),
            pl.BlockSpec((_RB, dc), lambda i: (i, 0)),
            pl.BlockSpec((_RB, dc), lambda i: (i, 0)),
            pl.BlockSpec((_RB, 1), lambda i: (i, 0)),
            pl.BlockSpec((1, d), lambda i: (0, 0)),
            pl.BlockSpec((d, d_out), lambda i: (0, 0)),
        ],
        out_specs=[
            pl.BlockSpec((_RB, dco), lambda i: (i, 0)),
            pl.BlockSpec((_RB, dco), lambda i: (i, 0)),
        ],
        out_shape=[
            jax.ShapeDtypeStruct((n, dco), jnp.bfloat16),
            jax.ShapeDtypeStruct((n, dco), jnp.bfloat16),
        ],
    )(acc_p, gl, gr, inv, b, W)


def _tc_final(acc_p, gl, gr, inv, b):
    n, dc = gl.shape
    d = 2 * dc
    return pl.pallas_call(
        _tc_final_body,
        grid=(n // _RB,),
        in_specs=[
            pl.BlockSpec((_NC, _RB, dc), lambda i: (0, i, 0)),
            pl.BlockSpec((_RB, dc), lambda i: (i, 0)),
            pl.BlockSpec((_RB, dc), lambda i: (i, 0)),
            pl.BlockSpec((_RB, 1), lambda i: (i, 0)),
            pl.BlockSpec((1, d), lambda i: (0, 0)),
        ],
        out_specs=pl.BlockSpec((_RB, d), lambda i: (i, 0)),
        out_shape=jax.ShapeDtypeStruct((n, d), jnp.float32),
    )(acc_p, gl, gr, inv, b)


def kernel(node_features, edge_index, W1, b1, W2, b2, W3, b3):
    n, d_in = node_features.shape
    e = edge_index.shape[1]
    epw = e // _NW
    ch = 125
    nchunk = (e // _NS) // ch

    src = edge_index[0].reshape(_NS, nchunk, ch)
    dst = edge_index[1].reshape(_NS, nchunk, ch)
    dst_flat = edge_index[1].reshape(_NW, epw)
    zeros_deg = jnp.zeros((_HALF * 16,), jnp.float32)
    zeros_h = jnp.zeros((_FULL, W1.shape[1] // 2), jnp.bfloat16)
    zeros_o = jnp.zeros((_FULL, W3.shape[1] // 2), jnp.bfloat16)

    deg_call = pl.kernel(
        _deg_body,
        out_type=jax.ShapeDtypeStruct((_NW, _FULL), jnp.float32),
        mesh=_sc_mesh(),
        scratch_types=[
            pltpu.VMEM((epw,), jnp.int32),
            pltpu.VMEM((_HALF * 16,), jnp.float32),
            pltpu.VMEM((_HALF,), jnp.float32),
        ],
        compiler_params=pltpu.CompilerParams(needs_layout_passes=False),
    )
    deg_p = deg_call(dst_flat, zeros_deg).T

    dh = W1.shape[1] // 2
    do = W3.shape[1] // 2
    acc_h = _make_acc_call(dh, nchunk, ch)
    acc_o = _make_acc_call(do, nchunk, ch)

    g1l, g1r, inv = _tc_first(deg_p, node_features, W1)
    a1 = acc_h(g1l, g1r, src, dst, zeros_h)
    g2l, g2r = _tc_mid(a1, g1l, g1r, inv, b1.reshape(1, -1), W2)
    a2 = acc_h(g2l, g2r, src, dst, zeros_h)
    g3l, g3r = _tc_mid(a2, g2l, g2r, inv, b2.reshape(1, -1), W3)
    a3 = acc_o(g3l, g3r, src, dst, zeros_o)
    return _tc_final(a3, g3l, g3r, inv, b3.reshape(1, -1))


# speedup vs baseline: 31.2514x; 1.0968x over previous
import jax
import jax.numpy as jnp
from jax import lax
from jax.experimental import pallas as pl
from jax.experimental.pallas import tpu as pltpu
from jax.experimental.pallas import tpu_sc as plsc

_NC = 2
_NS = 16
_NW = _NC * _NS

_HALF = 5120
_FULL = 2 * _HALF

_RB = 1000


def _sc_mesh():
    return plsc.VectorSubcoreMesh(core_axis_name="c", subcore_axis_name="s",
                                  num_cores=_NC, num_subcores=_NS)


def _deg_body(dst_hbm, zeros_hbm, out_hbm, dst_v, hist_v, deg_v):
    cid = lax.axis_index("c")
    sid = lax.axis_index("s")
    wid = sid * _NC + cid
    epw = dst_v.shape[0]

    pltpu.sync_copy(dst_hbm.at[wid], dst_v)

    lane_base = lax.iota(jnp.int32, 16) * _HALF
    ones = jnp.ones((16,), jnp.float32)

    for p in range(2):
        lo = p * _HALF
        pltpu.sync_copy(zeros_hbm, hist_v)

        def body(i, carry, lo=lo):
            dvec = dst_v[pl.ds(i * 16, 16)]
            m = (dvec >= lo) & (dvec < lo + _HALF)
            idx = jnp.where(m, dvec - lo, 0) + lane_base
            plsc.addupdate_scatter(hist_v, [idx], ones, mask=m)
            return carry

        lax.fori_loop(0, epw // 16, body, 0)

        def red(q, carry):
            s = hist_v[pl.ds(q * 16, 16)]
            for l in range(1, 16):
                s = s + hist_v[pl.ds(l * _HALF + q * 16, 16)]
            deg_v[pl.ds(q * 16, 16)] = s
            return carry

        lax.fori_loop(0, _HALF // 16, red, 0)
        pltpu.sync_copy(deg_v, out_hbm.at[wid, pl.ds(p * _HALF, _HALF)])


_NBUF = 4


def _acc_body(g_hbm, src_hbm, dst_hbm, zeros_hbm, out_hbm,
              src_v, dst_v, bufs, gsems, ssems, shared_acc):
    npad = shared_acc.shape[0]
    nchunk = src_v.shape[0]
    cid = lax.axis_index("c")
    sid = lax.axis_index("s")
    wid = sid * _NC + cid
    rpt = npad // _NS
    r0 = pl.multiple_of(sid * rpt, 8)

    pltpu.sync_copy(src_hbm.at[wid], src_v)
    pltpu.sync_copy(dst_hbm.at[wid], dst_v)
    pltpu.sync_copy(zeros_hbm.at[pl.ds(r0, rpt)], shared_acc.at[pl.ds(r0, rpt)])
    plsc.subcore_barrier()

    def gather(c, b):
        pltpu.async_copy(g_hbm.at[src_v.at[c]], bufs[b], gsems[b])

    def wait_gather(c, b):
        pltpu.make_async_copy(g_hbm.at[src_v.at[c]], bufs[b], gsems[b]).wait()

    def scatter(c, b):
        pltpu.async_copy(bufs[b], shared_acc.at[dst_v.at[c]], ssems[b],
                         add=True)

    def wait_scatter(c, b):
        pltpu.make_async_copy(bufs[b], shared_acc.at[dst_v.at[c]],
                              ssems[b]).wait()

    gather(0, 0)
    gather(1, 1)

    def outer(gidx, carry):
        for b in range(_NBUF):
            c = gidx * _NBUF + b
            wait_gather(c, b)
            scatter(c, b)

            @pl.when(c >= 2)
            def _(c=c, b=b):
                wait_scatter(c - 2, (b + 2) % _NBUF)

            @pl.when(c + 2 < nchunk)
            def _(c=c, b=b):
                gather(c + 2, (b + 2) % _NBUF)
        return carry

    lax.fori_loop(0, nchunk // _NBUF, outer, 0)
    wait_scatter(nchunk - 2, (nchunk - 2) % _NBUF)
    wait_scatter(nchunk - 1, (nchunk - 1) % _NBUF)

    plsc.subcore_barrier()
    pltpu.sync_copy(shared_acc.at[pl.ds(r0, rpt)],
                    out_hbm.at[cid, pl.ds(r0, rpt)])


def _make_acc_call(dc, nchunk, ch):
    return pl.kernel(
        _acc_body,
        out_type=jax.ShapeDtypeStruct((_NC, _FULL, dc), jnp.bfloat16),
        mesh=_sc_mesh(),
        scratch_types=[
            pltpu.VMEM((nchunk, ch), jnp.int32),
            pltpu.VMEM((nchunk, ch), jnp.int32),
            tuple(pltpu.VMEM((ch, dc), jnp.bfloat16) for _ in range(_NBUF)),
            tuple(pltpu.SemaphoreType.DMA for _ in range(_NBUF)),
            tuple(pltpu.SemaphoreType.DMA for _ in range(_NBUF)),
            pltpu.VMEM_SHARED((_FULL, dc), jnp.bfloat16),
        ],
        compiler_params=pltpu.CompilerParams(needs_layout_passes=False,
                                             use_tc_tiling_on_sc=False),
    )


def _tc_first_body(deg_ref, x_ref, w_ref, g_ref, inv_ref):
    deg = jnp.sum(deg_ref[...], axis=1) + 1.0
    inv = lax.rsqrt(deg)[:, None]
    inv_ref[...] = inv
    h = jnp.dot(x_ref[...] * inv, w_ref[...],
                preferred_element_type=jnp.float32)
    g_ref[...] = h.astype(jnp.bfloat16)


def _tc_mid_body(acc_ref, g_ref, inv_ref, b_ref, w_ref, o_ref):
    inv = inv_ref[...]
    s = (acc_ref[0].astype(jnp.float32) + acc_ref[1].astype(jnp.float32)
         + g_ref[...].astype(jnp.float32))
    x = jnp.maximum(inv * s + b_ref[...], 0.0)
    h = jnp.dot(x * inv, w_ref[...], preferred_element_type=jnp.float32)
    o_ref[...] = h.astype(jnp.bfloat16)


def _tc_final_body(acc_ref, g_ref, inv_ref, b_ref, o_ref):
    s = (acc_ref[0].astype(jnp.float32) + acc_ref[1].astype(jnp.float32)
         + g_ref[...].astype(jnp.float32))
    o_ref[...] = inv_ref[...] * s + b_ref[...]


def _tc_first(deg_p, x, W):
    n, d_in = x.shape
    d_out = W.shape[1]
    return pl.pallas_call(
        _tc_first_body,
        grid=(n // _RB,),
        in_specs=[
            pl.BlockSpec((_RB, _NW), lambda i: (i, 0)),
            pl.BlockSpec((_RB, d_in), lambda i: (i, 0)),
            pl.BlockSpec((d_in, d_out), lambda i: (0, 0)),
        ],
        out_specs=[
            pl.BlockSpec((_RB, d_out), lambda i: (i, 0)),
            pl.BlockSpec((_RB, 1), lambda i: (i, 0)),
        ],
        out_shape=[
            jax.ShapeDtypeStruct((n, d_out), jnp.bfloat16),
            jax.ShapeDtypeStruct((n, 1), jnp.float32),
        ],
    )(deg_p, x, W)


def _tc_mid(acc_p, g, inv, b, W):
    n, d = g.shape
    d_out = W.shape[1]
    return pl.pallas_call(
        _tc_mid_body,
        grid=(n // _RB,),
        in_specs=[
            pl.BlockSpec((_NC, _RB, d), lambda i: (0, i, 0)),
            pl.BlockSpec((_RB, d), lambda i: (i, 0)),
            pl.BlockSpec((_RB, 1), lambda i: (i, 0)),
            pl.BlockSpec((1, d), lambda i: (0, 0)),
            pl.BlockSpec((d, d_out), lambda i: (0, 0)),
        ],
        out_specs=pl.BlockSpec((_RB, d_out), lambda i: (i, 0)),
        out_shape=jax.ShapeDtypeStruct((n, d_out), jnp.bfloat16),
    )(acc_p, g, inv, b, W)


def _tc_final(acc_p, g, inv, b):
    n, d = g.shape
    return pl.pallas_call(
        _tc_final_body,
        grid=(n // _RB,),
        in_specs=[
            pl.BlockSpec((_NC, _RB, d), lambda i: (0, i, 0)),
            pl.BlockSpec((_RB, d), lambda i: (i, 0)),
            pl.BlockSpec((_RB, 1), lambda i: (i, 0)),
            pl.BlockSpec((1, d), lambda i: (0, 0)),
        ],
        out_specs=pl.BlockSpec((_RB, d), lambda i: (i, 0)),
        out_shape=jax.ShapeDtypeStruct((n, d), jnp.float32),
    )(acc_p, g, inv, b)


def kernel(node_features, edge_index, W1, b1, W2, b2, W3, b3):
    n, d_in = node_features.shape
    e = edge_index.shape[1]
    epw = e // _NW
    ch = 125
    nchunk = epw // ch

    src = edge_index[0].reshape(_NW, nchunk, ch)
    dst = edge_index[1].reshape(_NW, nchunk, ch)
    dst_flat = edge_index[1].reshape(_NW, epw)
    zeros_deg = jnp.zeros((_HALF * 16,), jnp.float32)
    zeros_h = jnp.zeros((_FULL, W1.shape[1]), jnp.bfloat16)
    zeros_o = jnp.zeros((_FULL, W3.shape[1]), jnp.bfloat16)

    deg_call = pl.kernel(
        _deg_body,
        out_type=jax.ShapeDtypeStruct((_NW, _FULL), jnp.float32),
        mesh=_sc_mesh(),
        scratch_types=[
            pltpu.VMEM((epw,), jnp.int32),
            pltpu.VMEM((_HALF * 16,), jnp.float32),
            pltpu.VMEM((_HALF,), jnp.float32),
        ],
        compiler_params=pltpu.CompilerParams(needs_layout_passes=False),
    )
    deg_p = deg_call(dst_flat, zeros_deg).T

    acc_h = _make_acc_call(W1.shape[1], nchunk, ch)
    acc_o = _make_acc_call(W3.shape[1], nchunk, ch)

    g1, inv = _tc_first(deg_p, node_features, W1)
    a1 = acc_h(g1, src, dst, zeros_h)
    g2 = _tc_mid(a1, g1, inv, b1.reshape(1, -1), W2)
    a2 = acc_h(g2, src, dst, zeros_h)
    g3 = _tc_mid(a2, g2, inv, b2.reshape(1, -1), W3)
    a3 = acc_o(g3, src, dst, zeros_o)
    return _tc_final(a3, g3, inv, b3.reshape(1, -1))


# speedup vs baseline: 33.9828x; 1.0874x over previous
import jax
import jax.numpy as jnp
from jax import lax
from jax.experimental import pallas as pl
from jax.experimental.pallas import tpu as pltpu
from jax.experimental.pallas import tpu_sc as plsc

_NC = 2
_NS = 16
_NW = _NC * _NS

_HALF = 5120
_FULL = 2 * _HALF
_LSTRIDE = _HALF + 1

_RB = 1000


def _sc_mesh():
    return plsc.VectorSubcoreMesh(core_axis_name="c", subcore_axis_name="s",
                                  num_cores=_NC, num_subcores=_NS)


def _deg_body(dst_hbm, zeros_hbm, out_hbm, dst_v, hist_v, deg_v):
    cid = lax.axis_index("c")
    sid = lax.axis_index("s")
    wid = sid * _NC + cid
    epw = dst_v.shape[0]

    pltpu.sync_copy(dst_hbm.at[wid], dst_v)

    lane_base = lax.iota(jnp.int32, 16) * _LSTRIDE
    ones = jnp.ones((16,), jnp.float32)

    for p in range(2):
        lo = p * _HALF
        pltpu.sync_copy(zeros_hbm, hist_v)

        def body(i, carry, lo=lo):
            dvec = dst_v[pl.ds(i * 16, 16)]
            m = (dvec >= lo) & (dvec < lo + _HALF)
            idx = jnp.where(m, dvec - lo, 0) + lane_base
            plsc.addupdate_scatter(hist_v, [idx], ones, mask=m)
            return carry

        lax.fori_loop(0, epw // 16, body, 0)

        def red(q, carry):
            s = hist_v[pl.ds(q * 16, 16)]
            for l in range(1, 16):
                s = s + hist_v[pl.ds(l * _LSTRIDE + q * 16, 16)]
            deg_v[pl.ds(q * 16, 16)] = s
            return carry

        lax.fori_loop(0, _HALF // 16, red, 0)
        pltpu.sync_copy(deg_v, out_hbm.at[wid, pl.ds(p * _HALF, _HALF)])


_NBUF = 8


def _acc_body(g_hbm, src_hbm, dst_hbm, zeros_hbm, out_hbm,
              src_v, dst_v, bufs, gsems, ssems, shared_acc):
    npad = shared_acc.shape[0]
    nchunk = src_v.shape[0]
    cid = lax.axis_index("c")
    sid = lax.axis_index("s")
    wid = sid * _NC + cid
    rpt = npad // _NS
    r0 = pl.multiple_of(sid * rpt, 8)

    pltpu.sync_copy(src_hbm.at[wid], src_v)
    pltpu.sync_copy(dst_hbm.at[wid], dst_v)
    pltpu.sync_copy(zeros_hbm.at[pl.ds(r0, rpt)], shared_acc.at[pl.ds(r0, rpt)])
    plsc.subcore_barrier()

    def gather(c, b):
        pltpu.async_copy(g_hbm.at[src_v.at[c]], bufs[b], gsems[b])

    def wait_gather(c, b):
        pltpu.make_async_copy(g_hbm.at[src_v.at[c]], bufs[b], gsems[b]).wait()

    def scatter(c, b):
        pltpu.async_copy(bufs[b], shared_acc.at[dst_v.at[c]], ssems[b],
                         add=True)

    def wait_scatter(c, b):
        pltpu.make_async_copy(bufs[b], shared_acc.at[dst_v.at[c]],
                              ssems[b]).wait()

    depth = _NBUF // 2
    for c in range(depth):
        gather(c, c)

    def outer(gidx, carry):
        for b in range(_NBUF):
            c = gidx * _NBUF + b
            wait_gather(c, b)
            scatter(c, b)

            @pl.when(c >= depth)
            def _(c=c, b=b):
                wait_scatter(c - depth, (b + depth) % _NBUF)

            @pl.when(c + depth < nchunk)
            def _(c=c, b=b):
                gather(c + depth, (b + depth) % _NBUF)
        return carry

    lax.fori_loop(0, nchunk // _NBUF, outer, 0)
    for c in range(nchunk - depth, nchunk):
        wait_scatter(c, c % _NBUF)

    plsc.subcore_barrier()
    pltpu.sync_copy(shared_acc.at[pl.ds(r0, rpt)],
                    out_hbm.at[cid, pl.ds(r0, rpt)])


def _make_acc_call(dc, nchunk, ch):
    return pl.kernel(
        _acc_body,
        out_type=jax.ShapeDtypeStruct((_NC, _FULL, dc), jnp.bfloat16),
        mesh=_sc_mesh(),
        scratch_types=[
            pltpu.VMEM((nchunk, ch), jnp.int32),
            pltpu.VMEM((nchunk, ch), jnp.int32),
            tuple(pltpu.VMEM((ch, dc), jnp.bfloat16) for _ in range(_NBUF)),
            tuple(pltpu.SemaphoreType.DMA for _ in range(_NBUF)),
            tuple(pltpu.SemaphoreType.DMA for _ in range(_NBUF)),
            pltpu.VMEM_SHARED((_FULL, dc), jnp.bfloat16),
        ],
        compiler_params=pltpu.CompilerParams(needs_layout_passes=False,
                                             use_tc_tiling_on_sc=False),
    )


def _tc_first_body(deg_ref, x_ref, w_ref, g_ref, inv_ref):
    deg = jnp.sum(deg_ref[...], axis=1) + 1.0
    inv = lax.rsqrt(deg)[:, None]
    inv_ref[...] = inv
    h = jnp.dot(x_ref[...] * inv, w_ref[...],
                preferred_element_type=jnp.float32)
    g_ref[...] = h.astype(jnp.bfloat16)


def _tc_mid_body(acc_ref, g_ref, inv_ref, b_ref, w_ref, o_ref):
    inv = inv_ref[...]
    s = (acc_ref[0].astype(jnp.float32) + acc_ref[1].astype(jnp.float32)
         + g_ref[...].astype(jnp.float32))
    x = jnp.maximum(inv * s + b_ref[...], 0.0)
    h = jnp.dot(x * inv, w_ref[...], preferred_element_type=jnp.float32)
    o_ref[...] = h.astype(jnp.bfloat16)


def _tc_final_body(acc_ref, g_ref, inv_ref, b_ref, o_ref):
    s = (acc_ref[0].astype(jnp.float32) + acc_ref[1].astype(jnp.float32)
         + g_ref[...].astype(jnp.float32))
    o_ref[...] = inv_ref[...] * s + b_ref[...]


def _tc_first(deg_p, x, W):
    n, d_in = x.shape
    d_out = W.shape[1]
    return pl.pallas_call(
        _tc_first_body,
        grid=(n // _RB,),
        in_specs=[
            pl.BlockSpec((_RB, _NW), lambda i: (i, 0)),
            pl.BlockSpec((_RB, d_in), lambda i: (i, 0)),
            pl.BlockSpec((d_in, d_out), lambda i: (0, 0)),
        ],
        out_specs=[
            pl.BlockSpec((_RB, d_out), lambda i: (i, 0)),
            pl.BlockSpec((_RB, 1), lambda i: (i, 0)),
        ],
        out_shape=[
            jax.ShapeDtypeStruct((n, d_out), jnp.bfloat16),
            jax.ShapeDtypeStruct((n, 1), jnp.float32),
        ],
    )(deg_p, x, W)


def _tc_mid(acc_p, g, inv, b, W):
    n, d = g.shape
    d_out = W.shape[1]
    return pl.pallas_call(
        _tc_mid_body,
        grid=(n // _RB,),
        in_specs=[
            pl.BlockSpec((_NC, _RB, d), lambda i: (0, i, 0)),
            pl.BlockSpec((_RB, d), lambda i: (i, 0)),
            pl.BlockSpec((_RB, 1), lambda i: (i, 0)),
            pl.BlockSpec((1, d), lambda i: (0, 0)),
            pl.BlockSpec((d, d_out), lambda i: (0, 0)),
        ],
        out_specs=pl.BlockSpec((_RB, d_out), lambda i: (i, 0)),
        out_shape=jax.ShapeDtypeStruct((n, d_out), jnp.bfloat16),
    )(acc_p, g, inv, b, W)


def _tc_final(acc_p, g, inv, b):
    n, d = g.shape
    return pl.pallas_call(
        _tc_final_body,
        grid=(n // _RB,),
        in_specs=[
            pl.BlockSpec((_NC, _RB, d), lambda i: (0, i, 0)),
            pl.BlockSpec((_RB, d), lambda i: (i, 0)),
            pl.BlockSpec((_RB, 1), lambda i: (i, 0)),
            pl.BlockSpec((1, d), lambda i: (0, 0)),
        ],
        out_specs=pl.BlockSpec((_RB, d), lambda i: (i, 0)),
        out_shape=jax.ShapeDtypeStruct((n, d), jnp.float32),
    )(acc_p, g, inv, b)


def kernel(node_features, edge_index, W1, b1, W2, b2, W3, b3):
    n, d_in = node_features.shape
    e = edge_index.shape[1]
    epw = e // _NW
    ch = 125
    nchunk = epw // ch

    src = edge_index[0].reshape(_NW, nchunk, ch)
    dst = edge_index[1].reshape(_NW, nchunk, ch)
    dst_flat = edge_index[1].reshape(_NW, epw)
    zeros_deg = jnp.zeros((_LSTRIDE * 16,), jnp.float32)
    zeros_h = jnp.zeros((_FULL, W1.shape[1]), jnp.bfloat16)
    zeros_o = jnp.zeros((_FULL, W3.shape[1]), jnp.bfloat16)

    deg_call = pl.kernel(
        _deg_body,
        out_type=jax.ShapeDtypeStruct((_NW, _FULL), jnp.float32),
        mesh=_sc_mesh(),
        scratch_types=[
            pltpu.VMEM((epw,), jnp.int32),
            pltpu.VMEM((_LSTRIDE * 16,), jnp.float32),
            pltpu.VMEM((_HALF,), jnp.float32),
        ],
        compiler_params=pltpu.CompilerParams(needs_layout_passes=False),
    )
    deg_p = deg_call(dst_flat, zeros_deg).T

    acc_h = _make_acc_call(W1.shape[1], nchunk, ch)
    acc_o = _make_acc_call(W3.shape[1], nchunk, ch)

    g1, inv = _tc_first(deg_p, node_features, W1)
    a1 = acc_h(g1, src, dst, zeros_h)
    g2 = _tc_mid(a1, g1, inv, b1.reshape(1, -1), W2)
    a2 = acc_h(g2, src, dst, zeros_h)
    g3 = _tc_mid(a2, g2, inv, b2.reshape(1, -1), W3)
    a3 = acc_o(g3, src, dst, zeros_o)
    return _tc_final(a3, g3, inv, b3.reshape(1, -1))
